# Initial kernel scaffold; baseline (speedup 1.0000x reference)
import jax
import jax.numpy as jnp
from jax.experimental import pallas as pl


def kernel(x, W):
    raise NotImplementedError("write your pallas kernel here")


# speedup vs baseline: 1.2149x; 1.2149x over previous
import jax
import jax.numpy as jnp
from jax.experimental import pallas as pl
from jax.experimental.pallas import tpu as pltpu

NUM_EXPERTS = 64
TOP_K = 8
HIDDEN = 4096
BATCH = 2
SEQ = 4096
TOKENS = BATCH * SEQ
TILE_T = 512


def _router_body(x_ref, w_ref, vals_ref, idx_ref, aux_ref, cnt_ref, psum_ref):
    i = pl.program_id(0)

    @pl.when(i == 0)
    def _init():
        cnt_ref[...] = jnp.zeros_like(cnt_ref)
        psum_ref[...] = jnp.zeros_like(psum_ref)

    x = x_ref[...]
    w = w_ref[...]
    logits = jax.lax.dot_general(
        x, w, (((1,), (1,)), ((), ())),
        preferred_element_type=jnp.float32)

    m = jnp.max(logits, axis=-1, keepdims=True)
    e = jnp.exp(logits - m)
    s = jnp.sum(e, axis=-1, keepdims=True)
    probs = e / s
    psum_ref[...] += jnp.sum(probs, axis=0, keepdims=True)

    iota = jax.lax.broadcasted_iota(jnp.int32, probs.shape, 1)
    work = probs
    vals_cols = []
    idx_cols = []
    for _ in range(TOP_K):
        mk = jnp.max(work, axis=-1, keepdims=True)
        ik = jnp.min(jnp.where(work == mk, iota, NUM_EXPERTS),
                     axis=-1, keepdims=True)
        vals_cols.append(mk)
        idx_cols.append(ik)
        work = jnp.where(iota == ik, -1.0, work)

    vals = jnp.concatenate(vals_cols, axis=1)
    idxs = jnp.concatenate(idx_cols, axis=1)
    vals_ref[...] = vals / jnp.sum(vals, axis=1, keepdims=True)
    idx_ref[...] = idxs

    cnt_ref[...] += jnp.sum(jnp.where(work < 0.0, 1.0, 0.0),
                            axis=0, keepdims=True)

    @pl.when(i == pl.num_programs(0) - 1)
    def _fin():
        aux = jnp.float32(NUM_EXPERTS) * jnp.sum(
            (cnt_ref[...] / jnp.float32(BATCH))
            * (psum_ref[...] / jnp.float32(TOKENS)))
        aux_ref[...] = jnp.reshape(aux, (1, 1))


def kernel(x, W):
    xt = x.reshape(TOKENS, HIDDEN)
    grid = TOKENS // TILE_T
    vals, idxs, aux = pl.pallas_call(
        _router_body,
        grid=(grid,),
        in_specs=[
            pl.BlockSpec((TILE_T, HIDDEN), lambda i: (i, 0)),
            pl.BlockSpec((NUM_EXPERTS, HIDDEN), lambda i: (0, 0)),
        ],
        out_specs=[
            pl.BlockSpec((TILE_T, TOP_K), lambda i: (i, 0)),
            pl.BlockSpec((TILE_T, TOP_K), lambda i: (i, 0)),
            pl.BlockSpec((1, 1), lambda i: (0, 0)),
        ],
        out_shape=[
            jax.ShapeDtypeStruct((TOKENS, TOP_K), jnp.float32),
            jax.ShapeDtypeStruct((TOKENS, TOP_K), jnp.int32),
            jax.ShapeDtypeStruct((1, 1), jnp.float32),
        ],
        scratch_shapes=[
            pltpu.VMEM((1, NUM_EXPERTS), jnp.float32),
            pltpu.VMEM((1, NUM_EXPERTS), jnp.float32),
        ],
        compiler_params=pltpu.CompilerParams(
            dimension_semantics=("arbitrary",),
        ),
    )(xt, W)
    return (vals.reshape(BATCH, SEQ, TOP_K),
            idxs.reshape(BATCH, SEQ, TOP_K),
            aux[0, 0])


# speedup vs baseline: 1.3536x; 1.1142x over previous
import jax
import jax.numpy as jnp
from jax.experimental import pallas as pl
from jax.experimental.pallas import tpu as pltpu

NUM_EXPERTS = 64
TOP_K = 8
HIDDEN = 4096
BATCH = 2
SEQ = 4096
TOKENS = BATCH * SEQ
TILE_T = 1024


def _router_body(x_ref, w_ref, vals_ref, idx_ref, aux_ref, cnt_ref, psum_ref):
    i = pl.program_id(0)

    @pl.when(i == 0)
    def _init():
        cnt_ref[...] = jnp.zeros_like(cnt_ref)
        psum_ref[...] = jnp.zeros_like(psum_ref)

    x = x_ref[...]
    w = w_ref[...]
    logits = jax.lax.dot_general(
        x, w, (((1,), (1,)), ((), ())),
        preferred_element_type=jnp.float32)

    m = jnp.max(logits, axis=-1, keepdims=True)
    e = jnp.exp(logits - m)
    s = jnp.sum(e, axis=-1, keepdims=True)
    probs = e / s
    psum_ref[...] += jnp.sum(probs, axis=0, keepdims=True)

    iota = jax.lax.broadcasted_iota(jnp.int32, probs.shape, 1)
    work = probs
    vals_cols = []
    idx_cols = []
    for _ in range(TOP_K):
        mk = jnp.max(work, axis=-1, keepdims=True)
        ik = jnp.min(jnp.where(work == mk, iota, NUM_EXPERTS),
                     axis=-1, keepdims=True)
        vals_cols.append(mk)
        idx_cols.append(ik)
        work = jnp.where(iota == ik, -1.0, work)

    vals = jnp.concatenate(vals_cols, axis=1)
    idxs = jnp.concatenate(idx_cols, axis=1)
    vals_ref[...] = vals / jnp.sum(vals, axis=1, keepdims=True)
    idx_ref[...] = idxs

    cnt_ref[...] += jnp.sum(jnp.where(work < 0.0, 1.0, 0.0),
                            axis=0, keepdims=True)

    @pl.when(i == pl.num_programs(0) - 1)
    def _fin():
        aux = jnp.float32(NUM_EXPERTS) * jnp.sum(
            (cnt_ref[...] / jnp.float32(BATCH))
            * (psum_ref[...] / jnp.float32(TOKENS)))
        aux_ref[...] = jnp.reshape(aux, (1, 1))


def kernel(x, W):
    xt = x.reshape(TOKENS, HIDDEN)
    grid = TOKENS // TILE_T
    vals, idxs, aux = pl.pallas_call(
        _router_body,
        grid=(grid,),
        in_specs=[
            pl.BlockSpec((TILE_T, HIDDEN), lambda i: (i, 0)),
            pl.BlockSpec((NUM_EXPERTS, HIDDEN), lambda i: (0, 0)),
        ],
        out_specs=[
            pl.BlockSpec((TILE_T, TOP_K), lambda i: (i, 0)),
            pl.BlockSpec((TILE_T, TOP_K), lambda i: (i, 0)),
            pl.BlockSpec((1, 1), lambda i: (0, 0)),
        ],
        out_shape=[
            jax.ShapeDtypeStruct((TOKENS, TOP_K), jnp.float32),
            jax.ShapeDtypeStruct((TOKENS, TOP_K), jnp.int32),
            jax.ShapeDtypeStruct((1, 1), jnp.float32),
        ],
        scratch_shapes=[
            pltpu.VMEM((1, NUM_EXPERTS), jnp.float32),
            pltpu.VMEM((1, NUM_EXPERTS), jnp.float32),
        ],
        compiler_params=pltpu.CompilerParams(
            dimension_semantics=("arbitrary",),
        ),
    )(xt, W)
    return (vals.reshape(BATCH, SEQ, TOP_K),
            idxs.reshape(BATCH, SEQ, TOP_K),
            aux[0, 0])
